# baseline (speedup 1.0000x reference)
import jax
import jax.numpy as jnp
from jax.experimental import pallas as pl
from jax.experimental.pallas import tpu as pltpu

_BM = 200


def _outer_kernel(x_ref, w_ref, adj_hbm, out_hbm, s_ref):
    s_ref[...] = jnp.dot(
        x_ref[...], w_ref[...], preferred_element_type=jnp.float32)

    n = adj_hbm.shape[0]
    f_out = s_ref.shape[1]

    def inner(adj_blk, out_blk):
        out_blk[...] = jnp.dot(
            adj_blk[...], s_ref[...], preferred_element_type=jnp.float32)

    pltpu.emit_pipeline(
        inner,
        grid=(n // _BM,),
        in_specs=[
            pl.BlockSpec((_BM, n), lambda i: (i, 0),
                         pipeline_mode=pl.Buffered(buffer_count=4)),
        ],
        out_specs=[pl.BlockSpec((_BM, f_out), lambda i: (i, 0))],
    )(adj_hbm, out_hbm)


def kernel(input, adj, weight):
    n, f_in = input.shape
    f_out = weight.shape[1]
    return pl.pallas_call(
        _outer_kernel,
        in_specs=[
            pl.BlockSpec((n, f_in), lambda: (0, 0)),
            pl.BlockSpec((f_in, f_out), lambda: (0, 0)),
            pl.BlockSpec(memory_space=pl.ANY),
        ],
        out_specs=pl.BlockSpec(memory_space=pl.ANY),
        out_shape=jax.ShapeDtypeStruct((n, f_out), jnp.float32),
        scratch_shapes=[pltpu.VMEM((n, f_out), jnp.float32)],
    )(input, weight, adj)


# speedup vs baseline: 1.0127x; 1.0127x over previous
import jax
import jax.numpy as jnp
from jax.experimental import pallas as pl
from jax.experimental.pallas import tpu as pltpu

_BM = 400


def _fused_kernel(x_ref, w_ref, adj_ref, out_ref, s_ref):
    i = pl.program_id(0)

    @pl.when(i == 0)
    def _():
        s_ref[...] = jnp.dot(
            x_ref[...],
            w_ref[...],
            preferred_element_type=jnp.float32,
        )

    out_ref[pl.ds(i * _BM, _BM), :] = jnp.dot(
        adj_ref[...],
        s_ref[...],
        preferred_element_type=jnp.float32,
    )


def kernel(input, adj, weight):
    n, f_in = input.shape
    f_out = weight.shape[1]
    return pl.pallas_call(
        _fused_kernel,
        grid=(pl.cdiv(n, _BM),),
        in_specs=[
            pl.BlockSpec((n, f_in), lambda i: (0, 0)),
            pl.BlockSpec((f_in, f_out), lambda i: (0, 0)),
            pl.BlockSpec((_BM, n), lambda i: (i, 0)),
        ],
        out_specs=pl.BlockSpec((n, f_out), lambda i: (0, 0)),
        out_shape=jax.ShapeDtypeStruct((n, f_out), jnp.float32),
        scratch_shapes=[pltpu.VMEM((n, f_out), jnp.float32)],
    )(input, weight, adj)
